# baseline (speedup 1.0000x reference)
import functools

import numpy as np
import jax
import jax.numpy as jnp
from jax import lax
from jax.experimental import pallas as pl
from jax.experimental.pallas import tpu as pltpu
from jax.experimental.pallas import tpu_sc as plsc

B = 4096
L = 200
D = 64
N_ROWS = B * L
NW = 32
PER_W = N_ROWS // NW
SUBG = 128
CHUNK = 512
NSUB = CHUNK // SUBG
N_CHUNKS = PER_W // CHUNK
PAIRS = N_CHUNKS // 2
BB = 64


def _positional_encoding():
    pos = np.arange(L, dtype=np.float32)[:, None]
    div = np.exp(np.arange(0, D, 2, dtype=np.float32) * (-np.log(10000.0) / D))
    pe = np.zeros((L, D), dtype=np.float32)
    pe[:, 0::2] = np.sin(pos * div)
    pe[:, 1::2] = np.cos(pos * div)
    return jnp.asarray(pe)


@functools.partial(
    pl.kernel,
    mesh=plsc.VectorSubcoreMesh(core_axis_name="c", subcore_axis_name="s"),
    compiler_params=pltpu.CompilerParams(use_tc_tiling_on_sc=False),
    out_type=jax.ShapeDtypeStruct((N_ROWS, D), jnp.float32),
    scratch_types=[
        pltpu.VMEM((PER_W,), jnp.int32),
        pltpu.VMEM((CHUNK, D), jnp.float32),
        pltpu.VMEM((CHUNK, D), jnp.float32),
        pltpu.SemaphoreType.DMA,
        pltpu.SemaphoreType.DMA,
    ],
)
def _sc_gather(seq_hbm, table_hbm, out_hbm, idx_v, rows0, rows1, sg0, sg1):
    nc = lax.axis_size("c")
    wid = lax.axis_index("s") * nc + lax.axis_index("c")
    row0 = wid * PER_W
    pltpu.sync_copy(seq_hbm.at[pl.ds(row0, PER_W)], idx_v)

    def issue_gather(c, rows_ref, sem):
        cc = lax.min(c, N_CHUNKS - 1)
        for j in range(NSUB):
            pltpu.async_copy(
                table_hbm.at[idx_v.at[pl.ds(cc * CHUNK + j * SUBG, SUBG)]],
                rows_ref.at[pl.ds(j * SUBG, SUBG)],
                sem,
            )

    def wait_gather(rows_ref, sem):
        pltpu.make_async_copy(
            out_hbm.at[pl.ds(0, CHUNK)], rows_ref, sem
        ).wait()

    def step(c, buf, nbuf, sem, nsem):
        issue_gather(c + 1, nbuf, nsem)
        wait_gather(buf, sem)
        pltpu.sync_copy(buf, out_hbm.at[pl.ds(row0 + c * CHUNK, CHUNK)])

    issue_gather(0, rows0, sg0)

    def pair_body(g, carry):
        step(2 * g, rows0, rows1, sg0, sg1)
        step(2 * g + 1, rows1, rows0, sg1, sg0)
        return carry

    lax.fori_loop(0, PAIRS, pair_body, 0)
    wait_gather(rows0, sg0)


LP = L // 2
N_PACK = N_ROWS // 2
PACK_W = N_PACK // NW
PCHUNK = 2 * LP
N_ACHUNKS = PACK_W // PCHUNK
APAIRS = N_ACHUNKS // 2
LANES = 16


@functools.partial(
    pl.kernel,
    mesh=plsc.VectorSubcoreMesh(core_axis_name="c", subcore_axis_name="s"),
    compiler_params=pltpu.CompilerParams(use_tc_tiling_on_sc=True),
    out_type=jax.ShapeDtypeStruct((N_ROWS, D), jnp.float32),
    scratch_types=[
        pltpu.VMEM((PCHUNK, 2 * D), jnp.float32),
        pltpu.VMEM((PCHUNK, 2 * D), jnp.float32),
        pltpu.VMEM((2 * PCHUNK, D), jnp.float32),
        pltpu.VMEM((LP, 2 * D), jnp.float32),
        pltpu.SemaphoreType.DMA,
        pltpu.SemaphoreType.DMA,
    ],
)
def _sc_add(x_hbm, pe_hbm, out_hbm, in0, in1, comp_v, pe_v, s0, s1):
    nc = lax.axis_size("c")
    wid = lax.axis_index("s") * nc + lax.axis_index("c")
    prow0 = wid * PACK_W
    pltpu.sync_copy(pe_hbm, pe_v)

    def issue_in(c, buf, sem):
        cc = lax.min(c, N_ACHUNKS - 1)
        pltpu.async_copy(x_hbm.at[pl.ds(prow0 + cc * PCHUNK, PCHUNK)], buf, sem)

    def wait_in(buf, sem):
        pltpu.make_async_copy(x_hbm.at[pl.ds(0, PCHUNK)], buf, sem).wait()

    def compute(buf):
        def row_body(r, carry):
            pes = [pe_v[r, pl.ds(dd * LANES, LANES)] for dd in range(2 * D // LANES)]
            for h in range(2):
                pr = r + h * LP
                for dd in range(2 * D // LANES):
                    f, fd = 2 * pr + dd // 4, (dd % 4) * LANES
                    comp_v[f, pl.ds(fd, LANES)] = (
                        buf[pr, pl.ds(dd * LANES, LANES)] + pes[dd]
                    )
            return carry

        lax.fori_loop(0, LP, row_body, 0, unroll=4)

    def step(c, buf, nbuf, sem, nsem):
        issue_in(c + 1, nbuf, nsem)
        wait_in(buf, sem)
        compute(buf)
        pltpu.sync_copy(
            comp_v, out_hbm.at[pl.ds(2 * (prow0 + c * PCHUNK), 2 * PCHUNK)]
        )

    issue_in(0, in0, s0)

    def pair_body(g, carry):
        step(2 * g, in0, in1, s0, s1)
        step(2 * g + 1, in1, in0, s1, s0)
        return carry

    lax.fori_loop(0, APAIRS, pair_body, 0)
    wait_in(in0, s0)


def kernel(sequence, table):
    assert sequence.shape == (B, L), sequence.shape
    assert table.shape == (100001, D), table.shape
    seq1d = sequence.reshape(-1).astype(jnp.int32)
    gathered = _sc_gather(seq1d, table)
    x128 = gathered.reshape(N_PACK, 2 * D)
    pe128 = _positional_encoding().reshape(LP, 2 * D)
    out = _sc_add(x128, pe128)
    return out.reshape(B, L, D)


# speedup vs baseline: 1.0675x; 1.0675x over previous
import functools

import numpy as np
import jax
import jax.numpy as jnp
from jax import lax
from jax.experimental import pallas as pl
from jax.experimental.pallas import tpu as pltpu
from jax.experimental.pallas import tpu_sc as plsc

B = 4096
L = 200
D = 64
N_ROWS = B * L
NW = 32
PER_W = N_ROWS // NW
SUBG = 128
CHUNK = 512
NSUB = CHUNK // SUBG
N_CHUNKS = PER_W // CHUNK
PAIRS = N_CHUNKS // 2
BB = 64


def _positional_encoding():
    pos = np.arange(L, dtype=np.float32)[:, None]
    div = np.exp(np.arange(0, D, 2, dtype=np.float32) * (-np.log(10000.0) / D))
    pe = np.zeros((L, D), dtype=np.float32)
    pe[:, 0::2] = np.sin(pos * div)
    pe[:, 1::2] = np.cos(pos * div)
    return jnp.asarray(pe)


@functools.partial(
    pl.kernel,
    mesh=plsc.VectorSubcoreMesh(core_axis_name="c", subcore_axis_name="s"),
    compiler_params=pltpu.CompilerParams(use_tc_tiling_on_sc=False),
    out_type=jax.ShapeDtypeStruct((N_ROWS, D), jnp.float32),
    scratch_types=[
        pltpu.VMEM((PER_W,), jnp.int32),
        pltpu.VMEM((CHUNK, D), jnp.float32),
        pltpu.VMEM((CHUNK, D), jnp.float32),
        pltpu.SemaphoreType.DMA,
        pltpu.SemaphoreType.DMA,
    ],
)
def _sc_gather(seq_hbm, table_hbm, out_hbm, idx_v, rows0, rows1, sg0, sg1):
    nc = lax.axis_size("c")
    wid = lax.axis_index("s") * nc + lax.axis_index("c")
    row0 = wid * PER_W
    pltpu.sync_copy(seq_hbm.at[pl.ds(row0, PER_W)], idx_v)

    def issue_gather(c, rows_ref, sem):
        cc = lax.min(c, N_CHUNKS - 1)
        for j in range(NSUB):
            pltpu.async_copy(
                table_hbm.at[idx_v.at[pl.ds(cc * CHUNK + j * SUBG, SUBG)]],
                rows_ref.at[pl.ds(j * SUBG, SUBG)],
                sem,
            )

    def wait_gather(rows_ref, sem):
        pltpu.make_async_copy(
            out_hbm.at[pl.ds(0, CHUNK)], rows_ref, sem
        ).wait()

    def step(c, buf, nbuf, sem, nsem):
        issue_gather(c + 1, nbuf, nsem)
        wait_gather(buf, sem)
        pltpu.sync_copy(buf, out_hbm.at[pl.ds(row0 + c * CHUNK, CHUNK)])

    issue_gather(0, rows0, sg0)

    def pair_body(g, carry):
        step(2 * g, rows0, rows1, sg0, sg1)
        step(2 * g + 1, rows1, rows0, sg1, sg0)
        return carry

    lax.fori_loop(0, PAIRS, pair_body, 0)
    wait_gather(rows0, sg0)


LP = L // 2
N_PACK = N_ROWS // 2
PACK_W = N_PACK // NW
PCHUNK = 2 * LP
HALF = LP
N_ACHUNKS = PACK_W // PCHUNK
APAIRS = (N_ACHUNKS - 2) // 2
LANES = 16


@functools.partial(
    pl.kernel,
    mesh=plsc.VectorSubcoreMesh(core_axis_name="c", subcore_axis_name="s"),
    compiler_params=pltpu.CompilerParams(use_tc_tiling_on_sc=True),
    out_type=jax.ShapeDtypeStruct((N_ROWS, D), jnp.float32),
    scratch_types=[
        pltpu.VMEM((PCHUNK, 2 * D), jnp.float32),
        pltpu.VMEM((PCHUNK, 2 * D), jnp.float32),
        pltpu.VMEM((2 * HALF, D), jnp.float32),
        pltpu.VMEM((2 * HALF, D), jnp.float32),
        pltpu.VMEM((LP, 2 * D), jnp.float32),
        pltpu.SemaphoreType.DMA,
        pltpu.SemaphoreType.DMA,
        pltpu.SemaphoreType.DMA,
        pltpu.SemaphoreType.DMA,
    ],
)
def _sc_add(x_hbm, pe_hbm, out_hbm, in0, in1, comp0, comp1, pe_v,
            s0, s1, sw0, sw1):
    nc = lax.axis_size("c")
    wid = lax.axis_index("s") * nc + lax.axis_index("c")
    prow0 = wid * PACK_W
    pltpu.sync_copy(pe_hbm, pe_v)
    ins = ((in0, s0), (in1, s1))
    comps = ((comp0, sw0), (comp1, sw1))

    def issue_in(c, buf, sem):
        cc = lax.min(c, N_ACHUNKS - 1)
        pltpu.async_copy(x_hbm.at[pl.ds(prow0 + cc * PCHUNK, PCHUNK)], buf, sem)

    def wait_in(buf, sem):
        pltpu.make_async_copy(x_hbm.at[pl.ds(0, PCHUNK)], buf, sem).wait()

    def compute_half(buf, h, comp_v):
        def row_body(r, carry):
            for dd in range(2 * D // LANES):
                f, fd = 2 * r + dd // 4, (dd % 4) * LANES
                comp_v[f, pl.ds(fd, LANES)] = (
                    buf[h * HALF + r, pl.ds(dd * LANES, LANES)]
                    + pe_v[r, pl.ds(dd * LANES, LANES)]
                )
            return carry

        lax.fori_loop(0, HALF, row_body, 0, unroll=4)

    def issue_write(c, h, comp_v, sem):
        pltpu.async_copy(
            comp_v,
            out_hbm.at[pl.ds(2 * (prow0 + c * PCHUNK + h * HALF), 2 * HALF)],
            sem,
        )

    def wait_write(comp_v, sem):
        pltpu.make_async_copy(
            comp_v, out_hbm.at[pl.ds(0, 2 * HALF)], sem
        ).wait()

    def step(c, par, wait_prev_write):
        buf, sem = ins[par]
        wait_in(buf, sem)
        issue_in(c + 1, *ins[1 - par])
        for h in range(2):
            comp_v, wsem = comps[h]
            if wait_prev_write:
                wait_write(comp_v, wsem)
            compute_half(buf, h, comp_v)
            issue_write(c, h, comp_v, wsem)

    issue_in(0, in0, s0)
    step(0, 0, False)
    step(1, 1, True)

    def pair_body(g, carry):
        c0 = 2 * g
        step(c0, 0, True)
        step(c0 + 1, 1, True)
        return carry

    lax.fori_loop(1, APAIRS + 1, pair_body, 0)
    wait_write(comp0, sw0)
    wait_write(comp1, sw1)
    wait_in(in0, s0)


def kernel(sequence, table):
    assert sequence.shape == (B, L), sequence.shape
    assert table.shape == (100001, D), table.shape
    seq1d = sequence.reshape(-1).astype(jnp.int32)
    gathered = _sc_gather(seq1d, table)
    x128 = gathered.reshape(N_PACK, 2 * D)
    pe128 = _positional_encoding().reshape(LP, 2 * D)
    out = _sc_add(x128, pe128)
    return out.reshape(B, L, D)


# speedup vs baseline: 1.1169x; 1.0463x over previous
import functools

import numpy as np
import jax
import jax.numpy as jnp
from jax import lax
from jax.experimental import pallas as pl
from jax.experimental.pallas import tpu as pltpu
from jax.experimental.pallas import tpu_sc as plsc

B = 4096
L = 200
D = 64
NW = 32
SEQ_PER_W = B // NW
SEQ_PER_CHUNK = 2
CHUNK = SEQ_PER_CHUNK * L
N_CHUNKS = SEQ_PER_W // SEQ_PER_CHUNK
PAIRS = N_CHUNKS // 2
SPLITS = ((0, 104), (104, 96))
LANES = 16


def _positional_encoding():
    pos = np.arange(L, dtype=np.float32)[:, None]
    div = np.exp(np.arange(0, D, 2, dtype=np.float32) * (-np.log(10000.0) / D))
    pe = np.zeros((L, D), dtype=np.float32)
    pe[:, 0::2] = np.sin(pos * div)
    pe[:, 1::2] = np.cos(pos * div)
    return jnp.asarray(pe)


@functools.partial(
    pl.kernel,
    mesh=plsc.VectorSubcoreMesh(core_axis_name="c", subcore_axis_name="s"),
    compiler_params=pltpu.CompilerParams(use_tc_tiling_on_sc=False),
    out_type=jax.ShapeDtypeStruct((B, L, D), jnp.float32),
    scratch_types=[
        pltpu.VMEM((SEQ_PER_W, L), jnp.int32),
        pltpu.VMEM((SEQ_PER_CHUNK, L, D), jnp.float32),
        pltpu.VMEM((SEQ_PER_CHUNK, L, D), jnp.float32),
        pltpu.VMEM((L, D), jnp.float32),
        pltpu.SemaphoreType.DMA,
        pltpu.SemaphoreType.DMA,
    ],
)
def _sc_embed(seq_hbm, pe_hbm, table_hbm, out_hbm,
              idx_v, rows0, rows1, pe_v, sg0, sg1):
    nc = lax.axis_size("c")
    wid = lax.axis_index("s") * nc + lax.axis_index("c")
    seq0 = wid * SEQ_PER_W
    pltpu.sync_copy(pe_hbm, pe_v)
    pltpu.sync_copy(seq_hbm.at[pl.ds(seq0, SEQ_PER_W)], idx_v)

    def issue_gather(c, rows_ref, sem):
        for s in range(SEQ_PER_CHUNK):
            for off, n in SPLITS:
                pltpu.async_copy(
                    table_hbm.at[idx_v.at[c * SEQ_PER_CHUNK + s, pl.ds(off, n)]],
                    rows_ref.at[s, pl.ds(off, n)],
                    sem,
                )

    def wait_gather(rows_ref, sem):
        pltpu.make_async_copy(
            out_hbm.at[pl.ds(0, SEQ_PER_CHUNK)], rows_ref, sem
        ).wait()

    def compute(rows_ref):
        def row_body(r, carry):
            for dd in range(D // LANES):
                sl = pl.ds(dd * LANES, LANES)
                pe_vec = pe_v[r, sl]
                for s in range(SEQ_PER_CHUNK):
                    rows_ref[s, r, sl] += pe_vec
            return carry

        lax.fori_loop(0, L, row_body, 0, unroll=8)

    def step(c, buf, nbuf, sem, nsem):
        issue_gather(lax.min(c + 1, N_CHUNKS - 1), nbuf, nsem)
        wait_gather(buf, sem)
        compute(buf)
        pltpu.sync_copy(
            buf, out_hbm.at[pl.ds(seq0 + c * SEQ_PER_CHUNK, SEQ_PER_CHUNK)]
        )

    issue_gather(0, rows0, sg0)

    def pair_body(g, carry):
        step(2 * g, rows0, rows1, sg0, sg1)
        step(2 * g + 1, rows1, rows0, sg1, sg0)
        return carry

    lax.fori_loop(0, PAIRS, pair_body, 0)
    wait_gather(rows0, sg0)


def kernel(sequence, table):
    assert sequence.shape == (B, L), sequence.shape
    assert table.shape[1] == D, table.shape
    pe = _positional_encoding()
    return _sc_embed(sequence.astype(jnp.int32), pe, table)
